# Initial kernel scaffold; baseline (speedup 1.0000x reference)
import jax
import jax.numpy as jnp
from jax.experimental import pallas as pl


def kernel(features, temporal_mask, temporal_indices, time_pos_encoding, W1, b1, W2, b2, W3, b3):
    raise NotImplementedError("write your pallas kernel here")


# speedup vs baseline: 4.2630x; 4.2630x over previous
import functools

import jax
import jax.numpy as jnp
from jax.experimental import pallas as pl

NS = 16384
F = 32
D = 128
N_TE = 6
TOTAL_TIME = 300
BLOCK = 512


def _dyrender_body(idx_ref, tpe_ref, w1b_ref, b1_ref, f_ref, mask_ref,
                   w1a_ref, w2_ref, b2_ref, w3_ref, b3_ref, out_ref):
    idx = idx_ref[0, :]
    cols = jax.lax.broadcasted_iota(jnp.int32, (F, TOTAL_TIME), 1)
    onehot = (idx[:, None] == cols).astype(jnp.float32)
    te = jnp.dot(onehot, tpe_ref[...], preferred_element_type=jnp.float32)
    c = jnp.dot(te, w1b_ref[...], preferred_element_type=jnp.float32) + b1_ref[...]
    a = jnp.dot(f_ref[...], w1a_ref[...], preferred_element_type=jnp.float32)
    h1 = jnp.maximum(a[:, None, :] + c[None, :, :], 0.0).reshape(BLOCK * F, D)
    h2 = jnp.maximum(
        jnp.dot(h1, w2_ref[...], preferred_element_type=jnp.float32) + b2_ref[...],
        0.0)
    o = jnp.dot(h2, w3_ref[...], preferred_element_type=jnp.float32) + b3_ref[0, 0]
    out_ref[...] = o.reshape(BLOCK, F) * mask_ref[...]


@functools.partial(jax.jit, static_argnames=())
def kernel(features, temporal_mask, temporal_indices, time_pos_encoding,
           W1, b1, W2, b2, W3, b3):
    idx2d = temporal_indices.astype(jnp.int32).reshape(1, F)
    maskf = temporal_mask.astype(jnp.float32)
    w1a = W1[:D, :]
    w1b = W1[D:, :]
    b1r = b1.reshape(1, D)
    b2r = b2.reshape(1, D)
    b3r = b3.reshape(1, 1)

    grid = (NS // BLOCK,)
    rep = lambda i: (0, 0)
    out = pl.pallas_call(
        _dyrender_body,
        grid=grid,
        in_specs=[
            pl.BlockSpec((1, F), rep),
            pl.BlockSpec((TOTAL_TIME, N_TE), rep),
            pl.BlockSpec((N_TE, D), rep),
            pl.BlockSpec((1, D), rep),
            pl.BlockSpec((BLOCK, D), lambda i: (i, 0)),
            pl.BlockSpec((BLOCK, F), lambda i: (i, 0)),
            pl.BlockSpec((D, D), rep),
            pl.BlockSpec((D, D), rep),
            pl.BlockSpec((1, D), rep),
            pl.BlockSpec((D, 1), rep),
            pl.BlockSpec((1, 1), rep),
        ],
        out_specs=pl.BlockSpec((BLOCK, F), lambda i: (i, 0)),
        out_shape=jax.ShapeDtypeStruct((NS, F), jnp.float32),
    )(idx2d, time_pos_encoding, w1b, b1r, features, maskf,
      w1a, W2, b2r, W3, b3r)
    return out


# speedup vs baseline: 6.4045x; 1.5023x over previous
import functools

import jax
import jax.numpy as jnp
from jax.experimental import pallas as pl

NS = 16384
F = 32
D = 128
N_TE = 6
TOTAL_TIME = 300
BLOCK = 512


def _dyrender_body(idx_ref, tpe_ref, w1b_ref, b1_ref, f_ref, mask_ref,
                   w1a_ref, w2_ref, b2_ref, w3s_ref, b3_ref, out_ref):
    idx = idx_ref[0, :]
    cols = jax.lax.broadcasted_iota(jnp.int32, (F, TOTAL_TIME), 1)
    onehot = (idx[:, None] == cols).astype(jnp.float32)
    te = jnp.dot(onehot, tpe_ref[...], preferred_element_type=jnp.float32)
    c = jnp.dot(te, w1b_ref[...], preferred_element_type=jnp.float32) + b1_ref[...]
    a = jnp.dot(f_ref[...], w1a_ref[...], preferred_element_type=jnp.float32)
    b2 = b2_ref[...]
    w2 = w2_ref[...]
    h2_tiles = []
    for f in range(F):
        h1 = jnp.maximum(a + c[f:f + 1, :], 0.0)
        z2 = jnp.dot(h1, w2, preferred_element_type=jnp.float32)
        h2_tiles.append(jnp.maximum(z2 + b2, 0.0))
    h2x = jnp.concatenate(h2_tiles, axis=1)
    o = jnp.dot(h2x, w3s_ref[...], preferred_element_type=jnp.float32)
    out_ref[...] = (o + b3_ref[0, 0]) * mask_ref[...]


@functools.partial(jax.jit, static_argnames=())
def kernel(features, temporal_mask, temporal_indices, time_pos_encoding,
           W1, b1, W2, b2, W3, b3):
    idx2d = temporal_indices.astype(jnp.int32).reshape(1, F)
    maskf = temporal_mask.astype(jnp.float32)
    w1a = W1[:D, :]
    w1b = W1[D:, :]
    b1r = b1.reshape(1, D)
    b2r = b2.reshape(1, D)
    b3r = b3.reshape(1, 1)
    w3s = jnp.kron(jnp.eye(F, dtype=jnp.float32), W3)

    grid = (NS // BLOCK,)
    rep = lambda i: (0, 0)
    out = pl.pallas_call(
        _dyrender_body,
        grid=grid,
        in_specs=[
            pl.BlockSpec((1, F), rep),
            pl.BlockSpec((TOTAL_TIME, N_TE), rep),
            pl.BlockSpec((N_TE, D), rep),
            pl.BlockSpec((1, D), rep),
            pl.BlockSpec((BLOCK, D), lambda i: (i, 0)),
            pl.BlockSpec((BLOCK, F), lambda i: (i, 0)),
            pl.BlockSpec((D, D), rep),
            pl.BlockSpec((D, D), rep),
            pl.BlockSpec((1, D), rep),
            pl.BlockSpec((F * D, F), rep),
            pl.BlockSpec((1, 1), rep),
        ],
        out_specs=pl.BlockSpec((BLOCK, F), lambda i: (i, 0)),
        out_shape=jax.ShapeDtypeStruct((NS, F), jnp.float32),
    )(idx2d, time_pos_encoding, w1b, b1r, features, maskf,
      w1a, W2, b2r, w3s, b3r)
    return out
